# Initial kernel scaffold; baseline (speedup 1.0000x reference)
import jax
import jax.numpy as jnp
from jax.experimental import pallas as pl


def kernel(x, edge_index, edge_attr, W1a, b1a, W2a, b2a, W3a, g1, be1, W1b, b1b, W2b, b2b, W3b, g2, be2):
    raise NotImplementedError("write your pallas kernel here")


# speedup vs baseline: 1.0463x; 1.0463x over previous
import functools

import jax
import jax.numpy as jnp
from jax import lax
from jax.experimental import pallas as pl
from jax.experimental.pallas import tpu as pltpu
from jax.experimental.pallas import tpu_sc as plsc

_N = 10000
_E = 160000
_NF = 32
_DE = 4
_EPS = 1e-5

_NC = 2
_NS = 16
_NW = _NC * _NS
_PW = _E // _NW
_CH = 128
_NFULL = _PW // _CH
_TAIL = _PW - _NFULL * _CH
_NT = _N // _NS

_ET = 640


def _sc_mesh():
    return plsc.VectorSubcoreMesh(core_axis_name="c", subcore_axis_name="s")


def _gather(table, idx):
    @functools.partial(
        pl.kernel,
        mesh=_sc_mesh(),
        compiler_params=pltpu.CompilerParams(use_tc_tiling_on_sc=False),
        out_type=jax.ShapeDtypeStruct((_E, _NF), jnp.float32),
        scratch_types=[
            pltpu.VMEM((_CH,), jnp.int32),
            pltpu.VMEM((_CH, _NF), jnp.float32),
            pltpu.VMEM((_TAIL,), jnp.int32),
            pltpu.VMEM((_TAIL, _NF), jnp.float32),
            pltpu.SemaphoreType.DMA,
        ],
    )
    def k(table_hbm, idx_hbm, out_hbm, idx_v, rows_v, idxt_v, rowst_v, sem):
        wid = lax.axis_index("s") * _NC + lax.axis_index("c")
        base = wid * _PW

        def body(j, carry):
            off = base + j * _CH
            pltpu.sync_copy(idx_hbm.at[pl.ds(off, _CH)], idx_v)
            pltpu.async_copy(table_hbm.at[idx_v], rows_v, sem).wait()
            pltpu.sync_copy(rows_v, out_hbm.at[pl.ds(off, _CH), :])
            return carry

        lax.fori_loop(0, _NFULL, body, 0)
        offt = base + _NFULL * _CH
        pltpu.sync_copy(idx_hbm.at[pl.ds(offt, _TAIL)], idxt_v)
        pltpu.async_copy(table_hbm.at[idxt_v], rowst_v, sem).wait()
        pltpu.sync_copy(rowst_v, out_hbm.at[pl.ds(offt, _TAIL), :])

    return k(table, idx)


def _scatter(msg, dst, zeros, ones, with_deg):
    out_type = [jax.ShapeDtypeStruct((_NC, _N, _NF), jnp.float32)]
    scratch = [
        pltpu.VMEM((_CH,), jnp.int32),
        pltpu.VMEM((_CH, _NF), jnp.float32),
        pltpu.VMEM((_TAIL,), jnp.int32),
        pltpu.VMEM((_TAIL, _NF), jnp.float32),
        pltpu.VMEM_SHARED((_N, _NF), jnp.float32),
    ]
    if with_deg:
        out_type.append(jax.ShapeDtypeStruct((_NC, _N, _NF), jnp.float32))
        scratch += [
            pltpu.VMEM((_CH, _NF), jnp.float32),
            pltpu.VMEM((_TAIL, _NF), jnp.float32),
            pltpu.VMEM_SHARED((_N, _NF), jnp.float32),
        ]

    @functools.partial(
        pl.kernel,
        mesh=_sc_mesh(),
        compiler_params=pltpu.CompilerParams(use_tc_tiling_on_sc=False),
        out_type=tuple(out_type),
        scratch_types=scratch,
    )
    def k(msg_hbm, dst_hbm, zeros_hbm, ones_hbm, *refs):
        if with_deg:
            (agg_hbm, deg_hbm, idx_v, rows_v, idxt_v, rowst_v, acc_sh,
             ones_v, onest_v, dacc_sh) = refs
        else:
            agg_hbm, idx_v, rows_v, idxt_v, rowst_v, acc_sh = refs
        c = lax.axis_index("c")
        s = lax.axis_index("s")
        wid = s * _NC + c
        nbase = s * _NT
        pltpu.sync_copy(zeros_hbm.at[pl.ds(nbase, _NT), :],
                        acc_sh.at[pl.ds(nbase, _NT), :])
        if with_deg:
            pltpu.sync_copy(zeros_hbm.at[pl.ds(nbase, _NT), :],
                            dacc_sh.at[pl.ds(nbase, _NT), :])
            pltpu.sync_copy(ones_hbm, ones_v)
            pltpu.sync_copy(ones_hbm.at[pl.ds(0, _TAIL), :], onest_v)
        plsc.subcore_barrier()

        base = wid * _PW

        def body(j, carry):
            off = base + j * _CH
            pltpu.sync_copy(dst_hbm.at[pl.ds(off, _CH)], idx_v)
            pltpu.sync_copy(msg_hbm.at[pl.ds(off, _CH), :], rows_v)
            pltpu.sync_copy(rows_v, acc_sh.at[idx_v], add=True)
            if with_deg:
                pltpu.sync_copy(ones_v, dacc_sh.at[idx_v], add=True)
            return carry

        lax.fori_loop(0, _NFULL, body, 0)
        offt = base + _NFULL * _CH
        pltpu.sync_copy(dst_hbm.at[pl.ds(offt, _TAIL)], idxt_v)
        pltpu.sync_copy(msg_hbm.at[pl.ds(offt, _TAIL), :], rowst_v)
        pltpu.sync_copy(rowst_v, acc_sh.at[idxt_v], add=True)
        if with_deg:
            pltpu.sync_copy(onest_v, dacc_sh.at[idxt_v], add=True)

        plsc.subcore_barrier()
        pltpu.sync_copy(acc_sh.at[pl.ds(nbase, _NT), :],
                        agg_hbm.at[c, pl.ds(nbase, _NT), :])
        if with_deg:
            pltpu.sync_copy(dacc_sh.at[pl.ds(nbase, _NT), :],
                            deg_hbm.at[c, pl.ds(nbase, _NT), :])

    res = k(msg, dst, zeros, ones)
    return res if with_deg else res[0]


def _messages(ea, xj, W1, b1, W2, b2, W3):
    def body(ea_ref, xj_ref, W1_ref, b1_ref, W2_ref, b2_ref, W3_ref, out_ref):
        h1 = jnp.maximum(
            jnp.dot(ea_ref[...], W1_ref[...],
                    preferred_element_type=jnp.float32) + b1_ref[...], 0.0)
        h2 = jnp.maximum(
            jnp.dot(h1, W2_ref[...],
                    preferred_element_type=jnp.float32) + b2_ref[...], 0.0)
        theta = jnp.dot(h2, W3_ref[...],
                        preferred_element_type=jnp.float32)
        xj = xj_ref[...]
        acc = xj[:, 0:1] * theta[:, 0:_NF]
        for i in range(1, _NF):
            acc = acc + xj[:, i:i + 1] * theta[:, i * _NF:(i + 1) * _NF]
        out_ref[...] = acc

    grid = _E // _ET
    return pl.pallas_call(
        body,
        grid=(grid,),
        in_specs=[
            pl.BlockSpec((_ET, _DE), lambda i: (i, 0)),
            pl.BlockSpec((_ET, _NF), lambda i: (i, 0)),
            pl.BlockSpec((_DE, 32), lambda i: (0, 0)),
            pl.BlockSpec((1, 32), lambda i: (0, 0)),
            pl.BlockSpec((32, 64), lambda i: (0, 0)),
            pl.BlockSpec((1, 64), lambda i: (0, 0)),
            pl.BlockSpec((64, _NF * _NF), lambda i: (0, 0)),
        ],
        out_specs=pl.BlockSpec((_ET, _NF), lambda i: (i, 0)),
        out_shape=jax.ShapeDtypeStruct((_E, _NF), jnp.float32),
    )(ea, xj, W1, b1.reshape(1, -1), W2, b2.reshape(1, -1), W3)


def _bn(aggp, degp, g, be, x=None):
    def body(aggp_ref, degp_ref, g_ref, be_ref, *refs):
        if x is None:
            out_ref, = refs
        else:
            x_ref, out_ref = refs
        aggp = aggp_ref[...]
        degp = degp_ref[...]
        agg = aggp[0] + aggp[1]
        deg = jnp.maximum(degp[0] + degp[1], 1.0)
        y = agg / deg
        m = jnp.mean(y, axis=0, keepdims=True)
        v = jnp.mean((y - m) ** 2, axis=0, keepdims=True)
        yn = (y - m) * lax.rsqrt(v + _EPS) * g_ref[...] + be_ref[...]
        if x is not None:
            yn = yn + x_ref[...]
        out_ref[...] = jnp.maximum(yn, 0.0)

    args = [aggp, degp, g.reshape(1, -1), be.reshape(1, -1)]
    if x is not None:
        args.append(x)
    return pl.pallas_call(
        body,
        out_shape=jax.ShapeDtypeStruct((_N, _NF), jnp.float32),
    )(*args)


def kernel(x, edge_index, edge_attr, W1a, b1a, W2a, b2a, W3a, g1, be1,
           W1b, b1b, W2b, b2b, W3b, g2, be2):
    src = edge_index[0]
    dst = edge_index[1]
    zeros = jnp.zeros((_N, _NF), jnp.float32)
    ones = jnp.ones((_CH, _NF), jnp.float32)

    xj = _gather(x, src)
    msg_a = _messages(edge_attr, xj, W1a, b1a, W2a, b2a, W3a)
    agg_a, degp = _scatter(msg_a, dst, zeros, ones, with_deg=True)
    h = _bn(agg_a, degp, g1, be1)

    hj = _gather(h, src)
    msg_b = _messages(edge_attr, hj, W1b, b1b, W2b, b2b, W3b)
    agg_b = _scatter(msg_b, dst, zeros, ones, with_deg=False)
    return _bn(agg_b, degp, g2, be2, x)


# speedup vs baseline: 1.0956x; 1.0471x over previous
import functools

import jax
import jax.numpy as jnp
from jax import lax
from jax.experimental import pallas as pl
from jax.experimental.pallas import tpu as pltpu
from jax.experimental.pallas import tpu_sc as plsc

_N = 10000
_E = 160000
_NF = 32
_DE = 4
_EPS = 1e-5

_NC = 2
_NS = 16
_NW = _NC * _NS
_CH = 128
_ROWS = _E // _CH
_RPW = _ROWS // _NW
_XTRA = _ROWS - _RPW * _NW
_NB = 3
_NG = _RPW // _NB
_NT = _N // _NS

_ET = 640


def _sc_mesh():
    return plsc.VectorSubcoreMesh(core_axis_name="c", subcore_axis_name="s")


def _gather(table, idx2d):
    @functools.partial(
        pl.kernel,
        mesh=_sc_mesh(),
        compiler_params=pltpu.CompilerParams(use_tc_tiling_on_sc=False),
        out_type=jax.ShapeDtypeStruct((_E, _NF), jnp.float32),
        scratch_types=[
            pltpu.VMEM((_RPW + 1, _CH), jnp.int32),
            pltpu.VMEM((_NB, _CH, _NF), jnp.float32),
        ] + [pltpu.SemaphoreType.DMA] * (2 * _NB),
    )
    def k(table_hbm, idx_hbm, out_hbm, idx_v, rows_v, *sems):
        sg, sw = sems[:_NB], sems[_NB:]
        wid = lax.axis_index("s") * _NC + lax.axis_index("c")
        rbase = wid * _RPW
        pltpu.sync_copy(idx_hbm.at[pl.ds(rbase, _RPW), :],
                        idx_v.at[pl.ds(0, _RPW), :])

        @pl.when(wid < _XTRA)
        def _():
            pltpu.sync_copy(idx_hbm.at[pl.ds(_RPW * _NW + wid, 1), :],
                            idx_v.at[pl.ds(_RPW, 1), :])

        def group(g, carry):
            gh = []
            for b in range(_NB):
                j = g * _NB + b
                gh.append(pltpu.async_copy(
                    table_hbm.at[idx_v.at[j]], rows_v.at[b], sg[b]))
            wh = []
            for b in range(_NB):
                j = g * _NB + b
                gh[b].wait()
                wh.append(pltpu.async_copy(
                    rows_v.at[b],
                    out_hbm.at[pl.ds((rbase + j) * _CH, _CH), :], sw[b]))
            for b in range(_NB):
                wh[b].wait()
            return carry

        lax.fori_loop(0, _NG, group, 0)

        @pl.when(wid < _XTRA)
        def _():
            r = _RPW * _NW + wid
            pltpu.async_copy(
                table_hbm.at[idx_v.at[_RPW]], rows_v.at[0], sg[0]).wait()
            pltpu.sync_copy(rows_v.at[0], out_hbm.at[pl.ds(r * _CH, _CH), :])

    return k(table, idx2d)


def _scatter(msg, dst2d, zeros, ones, with_deg):
    out_type = [jax.ShapeDtypeStruct((_NC, _N, _NF), jnp.float32)]
    scratch = [
        pltpu.VMEM((_RPW + 1, _CH), jnp.int32),
        pltpu.VMEM((_NB, _CH, _NF), jnp.float32),
        pltpu.VMEM_SHARED((_N, _NF), jnp.float32),
    ] + [pltpu.SemaphoreType.DMA] * (2 * _NB)
    if with_deg:
        out_type.append(jax.ShapeDtypeStruct((_NC, _N, _NF), jnp.float32))
        scratch += [
            pltpu.VMEM((_CH, _NF), jnp.float32),
            pltpu.VMEM_SHARED((_N, _NF), jnp.float32),
        ] + [pltpu.SemaphoreType.DMA] * _NB

    @functools.partial(
        pl.kernel,
        mesh=_sc_mesh(),
        compiler_params=pltpu.CompilerParams(use_tc_tiling_on_sc=False),
        out_type=tuple(out_type),
        scratch_types=scratch,
    )
    def k(msg_hbm, dst_hbm, zeros_hbm, ones_hbm, *refs):
        if with_deg:
            (agg_hbm, deg_hbm, idx_v, rows_v, acc_sh, *sems) = refs[:5 + 2 * _NB]
            ones_v, dacc_sh = refs[5 + 2 * _NB:5 + 2 * _NB + 2]
            sd = refs[5 + 2 * _NB + 2:]
        else:
            (agg_hbm, idx_v, rows_v, acc_sh, *sems) = refs
            ones_v = dacc_sh = sd = None
        sl, ss = sems[:_NB], sems[_NB:2 * _NB]
        c = lax.axis_index("c")
        s = lax.axis_index("s")
        wid = s * _NC + c
        nbase = s * _NT
        rbase = wid * _RPW
        pltpu.sync_copy(dst_hbm.at[pl.ds(rbase, _RPW), :],
                        idx_v.at[pl.ds(0, _RPW), :])
        pltpu.sync_copy(zeros_hbm.at[pl.ds(nbase, _NT), :],
                        acc_sh.at[pl.ds(nbase, _NT), :])
        if with_deg:
            pltpu.sync_copy(zeros_hbm.at[pl.ds(nbase, _NT), :],
                            dacc_sh.at[pl.ds(nbase, _NT), :])
            pltpu.sync_copy(ones_hbm, ones_v)

        @pl.when(wid < _XTRA)
        def _():
            pltpu.sync_copy(dst_hbm.at[pl.ds(_RPW * _NW + wid, 1), :],
                            idx_v.at[pl.ds(_RPW, 1), :])

        plsc.subcore_barrier()

        def group(g, carry):
            lh = []
            for b in range(_NB):
                j = g * _NB + b
                lh.append(pltpu.async_copy(
                    msg_hbm.at[pl.ds((rbase + j) * _CH, _CH), :],
                    rows_v.at[b], sl[b]))
            sh = []
            for b in range(_NB):
                j = g * _NB + b
                lh[b].wait()
                sh.append(pltpu.async_copy(
                    rows_v.at[b], acc_sh.at[idx_v.at[j]], ss[b], add=True))
                if with_deg:
                    sh.append(pltpu.async_copy(
                        ones_v, dacc_sh.at[idx_v.at[j]], sd[b], add=True))
            for h in sh:
                h.wait()
            return carry

        lax.fori_loop(0, _NG, group, 0)

        @pl.when(wid < _XTRA)
        def _():
            r = _RPW * _NW + wid
            pltpu.async_copy(msg_hbm.at[pl.ds(r * _CH, _CH), :],
                             rows_v.at[0], sl[0]).wait()
            pltpu.async_copy(rows_v.at[0], acc_sh.at[idx_v.at[_RPW]],
                             ss[0], add=True).wait()
            if with_deg:
                pltpu.async_copy(ones_v, dacc_sh.at[idx_v.at[_RPW]],
                                 sd[0], add=True).wait()

        plsc.subcore_barrier()
        pltpu.sync_copy(acc_sh.at[pl.ds(nbase, _NT), :],
                        agg_hbm.at[c, pl.ds(nbase, _NT), :])
        if with_deg:
            pltpu.sync_copy(dacc_sh.at[pl.ds(nbase, _NT), :],
                            deg_hbm.at[c, pl.ds(nbase, _NT), :])

    res = k(msg, dst2d, zeros, ones)
    return res if with_deg else res[0]


def _messages(ea, xj, W1, b1, W2, b2, W3):
    def body(ea_ref, xj_ref, W1_ref, b1_ref, W2_ref, b2_ref, W3_ref, out_ref):
        h1 = jnp.maximum(
            jnp.dot(ea_ref[...], W1_ref[...],
                    preferred_element_type=jnp.float32) + b1_ref[...], 0.0)
        h2 = jnp.maximum(
            jnp.dot(h1, W2_ref[...],
                    preferred_element_type=jnp.float32) + b2_ref[...], 0.0)
        theta = jnp.dot(h2, W3_ref[...],
                        preferred_element_type=jnp.float32)
        xj = xj_ref[...]
        acc = xj[:, 0:1] * theta[:, 0:_NF]
        for i in range(1, _NF):
            acc = acc + xj[:, i:i + 1] * theta[:, i * _NF:(i + 1) * _NF]
        out_ref[...] = acc

    grid = _E // _ET
    return pl.pallas_call(
        body,
        grid=(grid,),
        in_specs=[
            pl.BlockSpec((_ET, _DE), lambda i: (i, 0)),
            pl.BlockSpec((_ET, _NF), lambda i: (i, 0)),
            pl.BlockSpec((_DE, 32), lambda i: (0, 0)),
            pl.BlockSpec((1, 32), lambda i: (0, 0)),
            pl.BlockSpec((32, 64), lambda i: (0, 0)),
            pl.BlockSpec((1, 64), lambda i: (0, 0)),
            pl.BlockSpec((64, _NF * _NF), lambda i: (0, 0)),
        ],
        out_specs=pl.BlockSpec((_ET, _NF), lambda i: (i, 0)),
        out_shape=jax.ShapeDtypeStruct((_E, _NF), jnp.float32),
    )(ea, xj, W1, b1.reshape(1, -1), W2, b2.reshape(1, -1), W3)


def _bn(aggp, degp, g, be, x=None):
    def body(aggp_ref, degp_ref, g_ref, be_ref, *refs):
        if x is None:
            out_ref, = refs
        else:
            x_ref, out_ref = refs
        aggp = aggp_ref[...]
        degp = degp_ref[...]
        agg = aggp[0] + aggp[1]
        deg = jnp.maximum(degp[0] + degp[1], 1.0)
        y = agg / deg
        m = jnp.mean(y, axis=0, keepdims=True)
        v = jnp.mean((y - m) ** 2, axis=0, keepdims=True)
        yn = (y - m) * lax.rsqrt(v + _EPS) * g_ref[...] + be_ref[...]
        if x is not None:
            yn = yn + x_ref[...]
        out_ref[...] = jnp.maximum(yn, 0.0)

    args = [aggp, degp, g.reshape(1, -1), be.reshape(1, -1)]
    if x is not None:
        args.append(x)
    return pl.pallas_call(
        body,
        out_shape=jax.ShapeDtypeStruct((_N, _NF), jnp.float32),
    )(*args)


def kernel(x, edge_index, edge_attr, W1a, b1a, W2a, b2a, W3a, g1, be1,
           W1b, b1b, W2b, b2b, W3b, g2, be2):
    src = edge_index[0].reshape(_ROWS, _CH)
    dst = edge_index[1].reshape(_ROWS, _CH)
    zeros = jnp.zeros((_N, _NF), jnp.float32)
    ones = jnp.ones((_CH, _NF), jnp.float32)

    xj = _gather(x, src)
    msg_a = _messages(edge_attr, xj, W1a, b1a, W2a, b2a, W3a)
    agg_a, degp = _scatter(msg_a, dst, zeros, ones, with_deg=True)
    h = _bn(agg_a, degp, g1, be1)

    hj = _gather(h, src)
    msg_b = _messages(edge_attr, hj, W1b, b1b, W2b, b2b, W3b)
    agg_b = _scatter(msg_b, dst, zeros, ones, with_deg=False)
    return _bn(agg_b, degp, g2, be2, x)
